# baseline (speedup 1.0000x reference)
import jax
import jax.numpy as jnp
from jax.experimental import pallas as pl
from jax.experimental.pallas import tpu as pltpu

_B = 128
_V = 100000
_VB = 8192
_G = _VB // 128
_NBLK = (_V + _VB - 1) // _VB
_BIG = 2**30


def _body(x_ref, ids_ref, probs_ref, val_sc, idx_sc):
    i = pl.program_id(0)
    x = x_ref[...]
    probs_ref[...] = x
    xg = x.reshape(_B, _G, 128)

    def _update(xm):
        bval = jnp.max(xm, axis=1)
        giota = jax.lax.broadcasted_iota(jnp.int32, (_B, _G, 128), 1)
        bg = jnp.min(jnp.where(xm == bval[:, None, :], giota, _BIG), axis=1)
        bg = bg + i * _G

        @pl.when(i == 0)
        def _():
            val_sc[...] = bval
            idx_sc[...] = bg

        @pl.when(i > 0)
        def _():
            better = bval > val_sc[...]
            val_sc[...] = jnp.where(better, bval, val_sc[...])
            idx_sc[...] = jnp.where(better, bg, idx_sc[...])

    @pl.when(i < _NBLK - 1)
    def _():
        _update(xg)

    @pl.when(i == _NBLK - 1)
    def _():
        col3 = (jax.lax.broadcasted_iota(jnp.int32, (_B, _G, 128), 1) * 128
                + jax.lax.broadcasted_iota(jnp.int32, (_B, _G, 128), 2)
                + i * _VB)
        _update(jnp.where(col3 < _V, xg, -jnp.inf))

        vacc = val_sc[...]
        col = idx_sc[...] * 128 + jax.lax.broadcasted_iota(
            jnp.int32, (_B, 128), 1)
        rmax = jnp.max(vacc, axis=1, keepdims=True)
        ids_ref[...] = jnp.min(
            jnp.where(vacc == rmax, col, _BIG), axis=1, keepdims=True)


def kernel(logits):
    ids, probs = pl.pallas_call(
        _body,
        grid=(_NBLK,),
        in_specs=[pl.BlockSpec((_B, _VB), lambda i: (0, i))],
        out_specs=[
            pl.BlockSpec((_B, 1), lambda i: (0, 0)),
            pl.BlockSpec((_B, _VB), lambda i: (0, i)),
        ],
        out_shape=[
            jax.ShapeDtypeStruct((_B, 1), jnp.int32),
            jax.ShapeDtypeStruct((_B, _V), jnp.float32),
        ],
        scratch_shapes=[
            pltpu.VMEM((_B, 128), jnp.float32),
            pltpu.VMEM((_B, 128), jnp.int32),
        ],
    )(logits)
    return ids.reshape(_B), probs


# speedup vs baseline: 1.0556x; 1.0556x over previous
import jax
import jax.numpy as jnp
from jax.experimental import pallas as pl
from jax.experimental.pallas import tpu as pltpu

_B = 128
_V = 100000
_VB = 16384
_TAIL = _V % 128
_VMAIN = _V - _TAIL
_NCH = (_VMAIN + _VB - 1) // _VB
_SZ = [_VB] * (_NCH - 1) + [_VMAIN - _VB * (_NCH - 1)]
_OFF = [i * _VB for i in range(_NCH)]
_BIG = 2**30
_NBUF = 3


def _body(x_hbm, ids_ref, probs_hbm, b0, b1, b2, tbuf, vacc, iacc,
          sin, sout, stin, stout):
    bufs = [b0, b1, b2]

    def in_cp(c):
        s = bufs[c % _NBUF]
        return pltpu.make_async_copy(
            x_hbm.at[:, pl.ds(_OFF[c], _SZ[c])], s.at[:, pl.ds(0, _SZ[c])],
            sin.at[c % _NBUF])

    def out_cp(c):
        s = bufs[c % _NBUF]
        return pltpu.make_async_copy(
            s.at[:, pl.ds(0, _SZ[c])], probs_hbm.at[:, pl.ds(_OFF[c], _SZ[c])],
            sout.at[c % _NBUF])

    tail_in = pltpu.make_async_copy(
        x_hbm.at[:, pl.ds(_VMAIN, _TAIL)], tbuf, stin)
    tail_out = pltpu.make_async_copy(
        tbuf, probs_hbm.at[:, pl.ds(_VMAIN, _TAIL)], stout)

    tail_in.start()
    for j in range(_NBUF):
        in_cp(j).start()

    for c in range(_NCH):
        in_cp(c).wait()
        out_cp(c).start()

        x = bufs[c % _NBUF][:, pl.ds(0, _SZ[c])]
        col = jax.lax.broadcasted_iota(jnp.int32, x.shape, 1) + _OFF[c]
        bmax = jnp.max(x, axis=1, keepdims=True)
        bidx = jnp.min(jnp.where(x == bmax, col, _BIG), axis=1, keepdims=True)
        if c == 0:
            vacc[...] = bmax
            iacc[...] = bidx
        else:
            better = bmax > vacc[...]
            vacc[...] = jnp.where(better, bmax, vacc[...])
            iacc[...] = jnp.where(better, bidx, iacc[...])

        nxt = c + _NBUF
        if nxt < _NCH:
            out_cp(c).wait()
            in_cp(nxt).start()

    tail_in.wait()
    tail_out.start()
    xt = tbuf[...]
    colt = jax.lax.broadcasted_iota(jnp.int32, xt.shape, 1) + _VMAIN
    tmax = jnp.max(xt, axis=1, keepdims=True)
    tidx = jnp.min(jnp.where(xt == tmax, colt, _BIG), axis=1, keepdims=True)
    better = tmax > vacc[...]
    iacc[...] = jnp.where(better, tidx, iacc[...])

    for c in range(_NCH - _NBUF, _NCH):
        out_cp(c).wait()
    tail_out.wait()

    ids_ref[...] = iacc[...]


def kernel(logits):
    ids, probs = pl.pallas_call(
        _body,
        in_specs=[pl.BlockSpec(memory_space=pl.ANY)],
        out_specs=[
            pl.BlockSpec(memory_space=pltpu.VMEM),
            pl.BlockSpec(memory_space=pl.ANY),
        ],
        out_shape=[
            jax.ShapeDtypeStruct((_B, 1), jnp.int32),
            jax.ShapeDtypeStruct((_B, _V), jnp.float32),
        ],
        scratch_shapes=[
            pltpu.VMEM((_B, _VB), jnp.float32),
            pltpu.VMEM((_B, _VB), jnp.float32),
            pltpu.VMEM((_B, _VB), jnp.float32),
            pltpu.VMEM((_B, _TAIL), jnp.float32),
            pltpu.VMEM((_B, 1), jnp.float32),
            pltpu.VMEM((_B, 1), jnp.int32),
            pltpu.SemaphoreType.DMA((_NBUF,)),
            pltpu.SemaphoreType.DMA((_NBUF,)),
            pltpu.SemaphoreType.DMA,
            pltpu.SemaphoreType.DMA,
        ],
    )(logits)
    return ids.reshape(_B), probs
